# baseline (speedup 1.0000x reference)
import functools

import jax
import jax.numpy as jnp
from jax import lax
from jax.experimental import pallas as pl
from jax.experimental.pallas import tpu as pltpu
from jax.experimental.pallas import tpu_sc as plsc

_VOCAB = 10000
_MAXLEN = 150
_DIM = 32
_BATCH = 4096

_GD = 8
_GP = 4
_DPW = _DIM // _GD
_PPW = 38
_LANES = 16
_NV = _BATCH // _LANES


def _emb_body(xT_hbm, tokT_hbm, posT_hbm, out_hbm,
              tab_v, posT_v, posb_v, idx_v, obuf_v, isem, wsem):
    wid = lax.axis_index("s") * 2 + lax.axis_index("c")
    dgrp = lax.rem(wid, _GD)
    pgrp = wid // _GD
    d0 = dgrp * _DPW
    p0 = pgrp * _PPW
    dblk = dgrp // 2
    dsub0 = lax.rem(dgrp, 2) * _DPW

    tab_cp = pltpu.async_copy(tokT_hbm.at[pl.ds(d0, _DPW)], tab_v, isem[0])
    pltpu.sync_copy(posT_hbm, posT_v)

    def pos_body(pi, _):
        p = p0 + pi
        pc = jnp.minimum(p, _MAXLEN - 1)
        base = (pc // _LANES) * _LANES
        psplat = jnp.full((_LANES,), pc - base, dtype=jnp.int32)
        for dd in range(_DPW):
            pv = posT_v[d0 + dd, pl.ds(base, _LANES)]
            pos_s = jax.lax.gather(
                pv,
                psplat[:, None],
                jax.lax.GatherDimensionNumbers(
                    offset_dims=(),
                    collapsed_slice_dims=(0,),
                    start_index_map=(0,),
                ),
                (1,),
                mode=jax.lax.GatherScatterMode.PROMISE_IN_BOUNDS,
            )
            posb_v[pi, dd, pl.ds(0, _LANES)] = pos_s
        return 0

    lax.fori_loop(0, _PPW, pos_body, 0)
    tab_cp.wait()

    def stage(pi, buf):
        pc = jnp.minimum(p0 + pi, _MAXLEN - 1)
        pltpu.async_copy(xT_hbm.at[pc], idx_v[buf], isem[buf])

    def process(pi, buf):
        p = p0 + pi
        pltpu.make_async_copy(xT_hbm.at[0], idx_v[buf], isem[buf]).wait()

        @pl.when((pi >= 2) & (p - 2 < _MAXLEN))
        def _():
            for dd in range(_DPW):
                pltpu.make_async_copy(
                    obuf_v[buf].at[dd],
                    out_hbm.at[0, 0, slice(None), 0, slice(None)],
                    wsem[buf],
                ).wait()

        pos_row = [posb_v[pi, dd, pl.ds(0, _LANES)] for dd in range(_DPW)]

        @plsc.parallel_loop(0, _NV // 2, unroll=4)
        def v_body(g):
            iv32 = idx_v[buf][pl.ds(g * _LANES, _LANES)]
            bblk = g // 4
            off = lax.rem(g, 4) * 2 * _LANES
            iv_lo = jnp.bitwise_and(iv32, 0xFFFF)
            iv_hi = lax.shift_right_logical(iv32, 16)
            for half, iv in ((0, iv_lo), (1, iv_hi)):
                for dd in range(_DPW):
                    col = plsc.load_gather(tab_v.at[dd], [iv])
                    obuf_v[buf][dd, bblk, pl.ds(off + half * _LANES, _LANES)] = (
                        col + pos_row[dd]
                    )

        @pl.when(p < _MAXLEN)
        def _():
            for dd in range(_DPW):
                pltpu.async_copy(
                    obuf_v[buf].at[dd],
                    out_hbm.at[p, dblk, slice(None), dsub0 + dd, slice(None)],
                    wsem[buf],
                )

    stage(0, 0)
    stage(1, 1)

    def loop_body(i, _):
        for b in range(2):
            pi = 2 * i + b
            process(pi, b)

            @pl.when(pi + 2 < _PPW)
            def _():
                stage(pi + 2, b)
        return 0

    lax.fori_loop(0, _PPW // 2, loop_body, 0)

    for b in range(2):
        @pl.when(p0 + _PPW - 2 + b < _MAXLEN)
        def _(b=b):
            for dd in range(_DPW):
                pltpu.make_async_copy(
                    obuf_v[b].at[dd],
                    out_hbm.at[0, 0, slice(None), 0, slice(None)],
                    wsem[b],
                ).wait()


@jax.jit
def _emb_call(xT, tokT, posT):
    mesh = plsc.VectorSubcoreMesh(core_axis_name="c", subcore_axis_name="s")
    k = functools.partial(
        pl.kernel,
        mesh=mesh,
        out_type=jax.ShapeDtypeStruct(
            (_MAXLEN, _DIM // 8, _BATCH // 128, 8, 128), jnp.float32
        ),
        scratch_types=[
            pltpu.VMEM((_DPW, _VOCAB), jnp.float32),
            pltpu.VMEM((_DIM, 160), jnp.float32),
            pltpu.VMEM((_PPW, _DPW, _LANES), jnp.float32),
            [pltpu.VMEM((_BATCH // 2,), jnp.int32) for _ in range(2)],
            [pltpu.VMEM((_DPW, _BATCH // 128, 128), jnp.float32) for _ in range(2)],
            [pltpu.SemaphoreType.DMA for _ in range(2)],
            [pltpu.SemaphoreType.DMA for _ in range(2)],
        ],
        compiler_params=pltpu.CompilerParams(
            use_tc_tiling_on_sc=False, needs_layout_passes=False
        ),
    )(_emb_body)
    return k(xT, tokT, posT)


def kernel(x, token_table, pos_table):
    x16 = x.astype(jnp.int16).T.reshape(_MAXLEN, _BATCH // 32, 2, _LANES)
    x16 = x16.swapaxes(2, 3).reshape(_MAXLEN, _BATCH // 2, 2)
    xT = jax.lax.bitcast_convert_type(x16, jnp.int32)
    tokT = token_table.T
    posT = jnp.pad(pos_table.T, ((0, 0), (0, 10)))
    out = _emb_call(xT, tokT, posT)
    return out.transpose(2, 4, 0, 1, 3).reshape(_BATCH, _MAXLEN, _DIM)


# speedup vs baseline: 1.0202x; 1.0202x over previous
import functools

import jax
import jax.numpy as jnp
from jax import lax
from jax.experimental import pallas as pl
from jax.experimental.pallas import tpu as pltpu
from jax.experimental.pallas import tpu_sc as plsc

_VOCAB = 10000
_MAXLEN = 150
_DIM = 32
_BATCH = 4096

_GD = 8
_GP = 4
_DPW = _DIM // _GD
_PPW = 38
_LANES = 16
_NV = _BATCH // _LANES


def _emb_body(xT_hbm, tokT_hbm, posT_hbm, out_hbm,
              tab_v, posT_v, posb_v, idx_v, obuf_v, isem, wsem):
    wid = lax.axis_index("s") * 2 + lax.axis_index("c")
    dgrp = lax.rem(wid, _GD)
    pgrp = wid // _GD
    d0 = dgrp * _DPW
    p0 = pgrp * _PPW
    dblk = dgrp // 2
    dsub0 = lax.rem(dgrp, 2) * _DPW

    tab_cp = pltpu.async_copy(tokT_hbm.at[pl.ds(d0, _DPW)], tab_v, isem[0])
    pltpu.sync_copy(posT_hbm, posT_v)

    def pos_body(pi, _):
        p = p0 + pi
        pc = jnp.minimum(p, _MAXLEN - 1)
        base = (pc // _LANES) * _LANES
        psplat = jnp.full((_LANES,), pc - base, dtype=jnp.int32)
        for dd in range(_DPW):
            pv = posT_v[d0 + dd, pl.ds(base, _LANES)]
            pos_s = jax.lax.gather(
                pv,
                psplat[:, None],
                jax.lax.GatherDimensionNumbers(
                    offset_dims=(),
                    collapsed_slice_dims=(0,),
                    start_index_map=(0,),
                ),
                (1,),
                mode=jax.lax.GatherScatterMode.PROMISE_IN_BOUNDS,
            )
            posb_v[pi, dd, pl.ds(0, _LANES)] = pos_s
        return 0

    lax.fori_loop(0, _PPW, pos_body, 0)
    tab_cp.wait()

    def stage(pi, buf):
        pc = jnp.minimum(p0 + pi, _MAXLEN - 1)
        pltpu.async_copy(xT_hbm.at[pc], idx_v[buf], isem[buf])

    def process(pi, buf):
        p = p0 + pi
        pltpu.make_async_copy(xT_hbm.at[0], idx_v[buf], isem[buf]).wait()

        @pl.when((pi >= 2) & (p - 2 < _MAXLEN))
        def _():
            for dd in range(_DPW):
                pltpu.make_async_copy(
                    obuf_v[buf].at[dd],
                    out_hbm.at[0, 0, slice(None), 0, slice(None)],
                    wsem[buf],
                ).wait()

        pos_row = [posb_v[pi, dd, pl.ds(0, _LANES)] for dd in range(_DPW)]

        @plsc.parallel_loop(0, _NV, unroll=16)
        def v_body(v):
            iv = idx_v[buf][pl.ds(v * _LANES, _LANES)]
            bblk = v // 8
            off = lax.rem(v, 8) * _LANES
            for dd in range(_DPW):
                col = plsc.load_gather(tab_v.at[dd], [iv])
                obuf_v[buf][dd, bblk, pl.ds(off, _LANES)] = col + pos_row[dd]

        @pl.when(p < _MAXLEN)
        def _():
            for dd in range(_DPW):
                pltpu.async_copy(
                    obuf_v[buf].at[dd],
                    out_hbm.at[p, dblk, slice(None), dsub0 + dd, slice(None)],
                    wsem[buf],
                )

    stage(0, 0)
    stage(1, 1)

    def loop_body(i, _):
        for b in range(2):
            pi = 2 * i + b
            process(pi, b)

            @pl.when(pi + 2 < _PPW)
            def _():
                stage(pi + 2, b)
        return 0

    lax.fori_loop(0, _PPW // 2, loop_body, 0)

    for b in range(2):
        @pl.when(p0 + _PPW - 2 + b < _MAXLEN)
        def _(b=b):
            for dd in range(_DPW):
                pltpu.make_async_copy(
                    obuf_v[b].at[dd],
                    out_hbm.at[0, 0, slice(None), 0, slice(None)],
                    wsem[b],
                ).wait()


@jax.jit
def _emb_call(xT, tokT, posT):
    mesh = plsc.VectorSubcoreMesh(core_axis_name="c", subcore_axis_name="s")
    k = functools.partial(
        pl.kernel,
        mesh=mesh,
        out_type=jax.ShapeDtypeStruct(
            (_MAXLEN, _DIM // 8, _BATCH // 128, 8, 128), jnp.float32
        ),
        scratch_types=[
            pltpu.VMEM((_DPW, _VOCAB), jnp.float32),
            pltpu.VMEM((_DIM, 160), jnp.float32),
            pltpu.VMEM((_PPW, _DPW, _LANES), jnp.float32),
            [pltpu.VMEM((_BATCH,), jnp.int32) for _ in range(2)],
            [pltpu.VMEM((_DPW, _BATCH // 128, 128), jnp.float32) for _ in range(2)],
            [pltpu.SemaphoreType.DMA for _ in range(2)],
            [pltpu.SemaphoreType.DMA for _ in range(2)],
        ],
        compiler_params=pltpu.CompilerParams(
            use_tc_tiling_on_sc=False, needs_layout_passes=False
        ),
    )(_emb_body)
    return k(xT, tokT, posT)


def kernel(x, token_table, pos_table):
    xT = x.astype(jnp.int32).T
    tokT = token_table.T
    posT = jnp.pad(pos_table.T, ((0, 0), (0, 10)))
    out = _emb_call(xT, tokT, posT)
    return out.transpose(2, 4, 0, 1, 3).reshape(_BATCH, _MAXLEN, _DIM)
